# Initial kernel scaffold; baseline (speedup 1.0000x reference)
import jax
import jax.numpy as jnp
from jax.experimental import pallas as pl


def kernel(label, cls_ctx, token_prefix, token_suffix):
    raise NotImplementedError("write your pallas kernel here")


# speedup vs baseline: 1.0299x; 1.0299x over previous
import functools

import jax
import jax.numpy as jnp
from jax import lax
from jax.experimental import pallas as pl
from jax.experimental.pallas import tpu as pltpu
from jax.experimental.pallas import tpu_sc as plsc

N_CLS = 8
N_PRE = 5
N_SUF = 64
SEQ = 77
D = 512


def _sc_counts():
    try:
        info = plsc.get_sparse_core_info()
        return int(info.num_cores), int(info.num_subcores)
    except Exception:
        return 2, 16


def kernel(label, cls_ctx, token_prefix, token_suffix):
    B = label.shape[0]
    NC, NS = _sc_counts()
    NW = NC * NS
    bw = B // NW

    lab = label.astype(jnp.int32)
    idx_cls = lab[:, None] * N_CLS + jnp.arange(N_CLS, dtype=jnp.int32)
    idx_suf = lab[:, None] * N_SUF + jnp.arange(N_SUF, dtype=jnp.int32)
    cls2 = cls_ctx.reshape(cls_ctx.shape[0] * N_CLS, D)
    suf2 = token_suffix.reshape(token_suffix.shape[0] * N_SUF, D)
    pref = token_prefix.reshape(N_PRE, D)

    mesh = plsc.VectorSubcoreMesh(core_axis_name="c", subcore_axis_name="s")

    @functools.partial(
        pl.kernel,
        mesh=mesh,
        compiler_params=pltpu.CompilerParams(use_tc_tiling_on_sc=False),
        out_type=jax.ShapeDtypeStruct((B, SEQ, D), jnp.float32),
        scratch_types=[
            pltpu.VMEM((bw, N_CLS), jnp.int32),
            pltpu.VMEM((bw, N_SUF), jnp.int32),
            pltpu.VMEM((N_PRE, D), jnp.float32),
            pltpu.VMEM((N_CLS, D), jnp.float32),
            pltpu.VMEM((N_SUF, D), jnp.float32),
            pltpu.SemaphoreType.DMA,
        ],
    )
    def _assemble(cls_hbm, suf_hbm, pref_hbm, ic_hbm, is_hbm, out_hbm,
                  ic_v, is_v, pref_v, cc_v, suf_v, gsem):
        wid = lax.axis_index("s") * NC + lax.axis_index("c")
        base = wid * bw
        pltpu.sync_copy(ic_hbm.at[pl.ds(base, bw)], ic_v)
        pltpu.sync_copy(is_hbm.at[pl.ds(base, bw)], is_v)
        pltpu.sync_copy(pref_hbm, pref_v)

        def body(i, carry):
            b = base + i
            g1 = pltpu.async_copy(cls_hbm.at[ic_v.at[i]], cc_v, gsem)
            g2 = pltpu.async_copy(suf_hbm.at[is_v.at[i]], suf_v, gsem)
            pltpu.sync_copy(pref_v, out_hbm.at[b, pl.ds(0, N_PRE)])
            g1.wait()
            g2.wait()
            pltpu.sync_copy(cc_v, out_hbm.at[b, pl.ds(N_PRE, N_CLS)])
            pltpu.sync_copy(suf_v, out_hbm.at[b, pl.ds(N_PRE + N_CLS, N_SUF)])
            return carry

        lax.fori_loop(0, bw, body, 0)

    return _assemble(cls2, suf2, pref, idx_cls, idx_suf)


# speedup vs baseline: 1.5508x; 1.5058x over previous
import functools

import jax
import jax.numpy as jnp
from jax import lax
from jax.experimental import pallas as pl
from jax.experimental.pallas import tpu as pltpu
from jax.experimental.pallas import tpu_sc as plsc

N_CLS = 8
N_PRE = 5
N_SUF = 64
SEQ = 77
SEQ_PAD = 80
D = 512
CPB = 8


def _sc_counts():
    try:
        info = plsc.get_sparse_core_info()
        return int(info.num_cores), int(info.num_subcores)
    except Exception:
        return 2, 16


def _build_fused(token_prefix, cls_ctx, token_suffix):
    n_cls_total = cls_ctx.shape[0]

    def body(p_ref, c_ref, s_ref, o_ref):
        o_ref[:, 0:N_PRE] = jnp.broadcast_to(p_ref[...], (CPB, N_PRE, D))
        o_ref[:, N_PRE:N_PRE + N_CLS] = c_ref[...]
        o_ref[:, N_PRE + N_CLS:SEQ] = s_ref[...]
        o_ref[:, SEQ:SEQ_PAD] = jnp.zeros((CPB, SEQ_PAD - SEQ, D), jnp.float32)

    return pl.pallas_call(
        body,
        grid=(n_cls_total // CPB,),
        in_specs=[
            pl.BlockSpec((1, N_PRE, D), lambda i: (0, 0, 0)),
            pl.BlockSpec((CPB, N_CLS, D), lambda i: (i, 0, 0)),
            pl.BlockSpec((CPB, N_SUF, D), lambda i: (i, 0, 0)),
        ],
        out_specs=pl.BlockSpec((CPB, SEQ_PAD, D), lambda i: (i, 0, 0)),
        out_shape=jax.ShapeDtypeStruct((n_cls_total, SEQ_PAD, D), jnp.float32),
    )(token_prefix, cls_ctx, token_suffix)


def kernel(label, cls_ctx, token_prefix, token_suffix):
    B = label.shape[0]
    NC, NS = _sc_counts()
    NW = NC * NS
    bw = B // NW

    fused = _build_fused(token_prefix, cls_ctx, token_suffix)
    flat = fused.reshape(fused.shape[0] * SEQ_PAD, D)
    lab = label.astype(jnp.int32)
    idx_all = (lab[:, None] * SEQ_PAD
               + jnp.arange(SEQ_PAD, dtype=jnp.int32)).reshape(-1)

    mesh = plsc.VectorSubcoreMesh(core_axis_name="c", subcore_axis_name="s")

    @functools.partial(
        pl.kernel,
        mesh=mesh,
        out_type=jax.ShapeDtypeStruct((B, SEQ, D), jnp.float32),
        scratch_types=[
            pltpu.VMEM((bw * SEQ_PAD,), jnp.int32),
            pltpu.VMEM((SEQ_PAD, D), jnp.float32),
            pltpu.SemaphoreType.DMA,
            pltpu.SemaphoreType.DMA,
        ],
    )
    def _gather(flat_hbm, idx_hbm, out_hbm, idx_v, buf, gsem, wsem):
        wid = lax.axis_index("s") * NC + lax.axis_index("c")
        base = wid * bw
        pltpu.sync_copy(idx_hbm.at[pl.ds(base * SEQ_PAD, bw * SEQ_PAD)], idx_v)

        def body(j, carry):
            g = pltpu.async_copy(
                flat_hbm.at[idx_v.at[pl.ds(j * SEQ_PAD, SEQ_PAD)]], buf, gsem)
            g.wait()
            w = pltpu.async_copy(
                buf, out_hbm.at[base + j, pl.ds(0, SEQ_PAD)], wsem)
            w.wait()
            return carry

        lax.fori_loop(0, bw, body, 0)

    return _gather(flat, idx_all)
